# Initial kernel scaffold; baseline (speedup 1.0000x reference)
import jax
import jax.numpy as jnp
from jax.experimental import pallas as pl


def kernel(user_emb, item_emb, edge_index, adj_vals, W_gc, b_gc, W_bi, b_bi):
    raise NotImplementedError("write your pallas kernel here")


# speedup vs baseline: 3.5711x; 3.5711x over previous
import functools

import jax
import jax.numpy as jnp
from jax import lax
from jax.experimental import pallas as pl
from jax.experimental.pallas import tpu as pltpu
from jax.experimental.pallas import tpu_sc as plsc

N_NODES = 10000
EMB = 128
LAYERS = 3
E = 320000

_NC = 2
_NS = 16
_NW = _NC * _NS
_EW = E // _NW
_C = 80
_NCHUNK = _EW // _C
_RT = N_NODES // _NS
_RO = 1000
_NS_OUT = N_NODES // _RO


def _spmm_body(ego_hbm, src_hbm, dst_hbm, vals_hbm, zeros_hbm, out_hbm,
               src_v, dst_v, vals_v, rows_v, acc_sh, sem):
    cid = lax.axis_index("c")
    sid = lax.axis_index("s")
    wid = sid * _NC + cid

    pltpu.sync_copy(zeros_hbm, acc_sh.at[pl.ds(sid * _RT, _RT)])
    plsc.subcore_barrier()

    def chunk_body(i, carry):
        base = pl.multiple_of(wid * _EW + i * _C, _C)
        pltpu.sync_copy(src_hbm.at[pl.ds(base, _C)], src_v)
        pltpu.sync_copy(dst_hbm.at[pl.ds(base, _C)], dst_v)
        pltpu.sync_copy(vals_hbm.at[pl.ds(base, _C)], vals_v.at[pl.ds(0, _C)])
        pltpu.async_copy(ego_hbm.at[src_v], rows_v, sem).wait()

        def edge_body(e, carry2):
            v = vals_v[pl.ds(e, 16)][0]
            for j in range(EMB // 16):
                sl = pl.ds(j * 16, 16)
                rows_v[e, sl] = rows_v[e, sl] * v
            return carry2

        lax.fori_loop(0, _C, edge_body, 0, unroll=False)
        pltpu.sync_copy(rows_v, acc_sh.at[dst_v], add=True)
        return carry

    lax.fori_loop(0, _NCHUNK, chunk_body, 0, unroll=False)

    plsc.subcore_barrier()

    @pl.when(sid < _NS_OUT)
    def _copy_out():
        off = pl.multiple_of(sid * _RO, 8)
        pltpu.sync_copy(acc_sh.at[pl.ds(off, _RO)],
                        out_hbm.at[cid, pl.ds(off, _RO)])


_spmm = pl.kernel(
    _spmm_body,
    out_type=jax.ShapeDtypeStruct((_NC, N_NODES, EMB), jnp.float32),
    mesh=plsc.VectorSubcoreMesh(core_axis_name="c", subcore_axis_name="s"),
    scratch_types=[
        pltpu.VMEM((_C,), jnp.int32),
        pltpu.VMEM((_C,), jnp.int32),
        pltpu.VMEM((_C + 16,), jnp.float32),
        pltpu.VMEM((_C, EMB), jnp.float32),
        pltpu.VMEM_SHARED((N_NODES, EMB), jnp.float32),
        pltpu.SemaphoreType.DMA,
    ],
)

_BLK = 1000


def _dense_body(p0_ref, p1_ref, ego_ref, wg_ref, wb_ref, b_ref,
                ego_out_ref, norm_out_ref):
    side = p0_ref[...] + p1_ref[...]
    ego = ego_ref[...]
    x = jnp.dot(side, wg_ref[...], preferred_element_type=jnp.float32)
    x = x + jnp.dot(ego * side, wb_ref[...], preferred_element_type=jnp.float32)
    x = x + b_ref[...]
    y = jnp.where(x > 0, x, 0.2 * x)
    ego_out_ref[...] = y
    nrm = jnp.sqrt(jnp.sum(y * y, axis=1, keepdims=True))
    norm_out_ref[...] = y / jnp.maximum(nrm, 1e-12)


def _dense(p0, p1, ego, wg, wb, b):
    row_spec = pl.BlockSpec((_BLK, EMB), lambda i: (i, 0))
    return pl.pallas_call(
        _dense_body,
        grid=(N_NODES // _BLK,),
        in_specs=[
            row_spec, row_spec, row_spec,
            pl.BlockSpec((EMB, EMB), lambda i: (0, 0)),
            pl.BlockSpec((EMB, EMB), lambda i: (0, 0)),
            pl.BlockSpec((1, EMB), lambda i: (0, 0)),
        ],
        out_specs=[row_spec, row_spec],
        out_shape=[
            jax.ShapeDtypeStruct((N_NODES, EMB), jnp.float32),
            jax.ShapeDtypeStruct((N_NODES, EMB), jnp.float32),
        ],
    )(p0, p1, ego, wg, wb, b)


def kernel(user_emb, item_emb, edge_index, adj_vals, W_gc, b_gc, W_bi, b_bi):
    ego0 = jnp.concatenate([user_emb, item_emb], axis=0)
    src = edge_index[1]
    dst = edge_index[0]
    zeros = jnp.zeros((_RT, EMB), jnp.float32)
    b_tot = b_gc + b_bi
    outs = [ego0]
    ego = ego0
    for k in range(LAYERS):
        parts = _spmm(ego, src, dst, adj_vals, zeros)
        ego, norm = _dense(parts[0], parts[1], ego, W_gc[k], W_bi[k], b_tot[k])
        outs.append(norm)
    return jnp.concatenate(outs, axis=1)
